# Initial kernel scaffold; baseline (speedup 1.0000x reference)
import functools

import jax
import jax.numpy as jnp
from jax.experimental import pallas as pl
from jax.experimental.pallas import tpu as pltpu

NUM_SEG = 8
TQ = 256
TK = 512
NEG = -1e30


def _attn_mlp_kernel(kv_t0_ref, kv_t1_ref, size_ref,
                     q_ref, kv_ref, qb_ref, kvb_ref,
                     w1t_ref, b1_ref, w2t_ref, b2_ref,
                     o_ref):
    i = pl.program_id(0)
    q = q_ref[...]
    qb = qb_ref[0, pl.ds(i * TQ, TQ)]
    qb_col = jnp.reshape(qb, (TQ, 1))

    t0 = kv_t0_ref[i]
    t1 = kv_t1_ref[i]

    m0 = jnp.full((TQ, 1), NEG, jnp.float32)
    l0 = jnp.zeros((TQ, 1), jnp.float32)
    acc0 = jnp.zeros_like(q)

    def body(t, carry):
        m, l, acc = carry
        kv = kv_ref[pl.ds(t * TK, TK), :]
        kvb = kvb_ref[0, pl.ds(t * TK, TK)]
        s = jax.lax.dot_general(q, kv, (((1,), (1,)), ((), ())),
                                preferred_element_type=jnp.float32)
        mask = qb_col == kvb[None, :]
        s = jnp.where(mask, s, NEG)
        m_new = jnp.maximum(m, jnp.max(s, axis=1, keepdims=True))
        p = jnp.where(mask, jnp.exp(s - m_new), 0.0)
        alpha = jnp.exp(m - m_new)
        l = l * alpha + jnp.sum(p, axis=1, keepdims=True)
        acc = acc * alpha + jax.lax.dot_general(
            p, kv, (((1,), (0,)), ((), ())),
            preferred_element_type=jnp.float32)
        return m_new, l, acc

    m, l, acc = jax.lax.fori_loop(t0, t1, body, (m0, l0, acc0))

    out = acc * jnp.where(l > 0.0, 1.0 / jnp.where(l > 0.0, l, 1.0), 0.0)
    res = out + q
    res = jnp.where(qb_col < size_ref[0], res, 0.0)

    h = jax.lax.dot_general(res, w1t_ref[...], (((1,), (0,)), ((), ())),
                            preferred_element_type=jnp.float32)
    h = jnp.maximum(h + b1_ref[...], 0.0)
    y = jax.lax.dot_general(h, w2t_ref[...], (((1,), (0,)), ((), ())),
                            preferred_element_type=jnp.float32)
    o_ref[...] = y + b2_ref[...] + res


@functools.partial(jax.jit, static_argnames=("interpret",))
def _cross_side(q, qb, kv, kvb, off_kv, size, w1t, b1, w2t, b2,
                interpret=False):
    n, d = q.shape
    nq = n // TQ
    qb2 = qb.reshape(nq, TQ)
    seg_lo = qb2[:, 0]
    seg_hi = qb2[:, -1]
    kv_t0 = (off_kv[seg_lo] // TK).astype(jnp.int32)
    kv_t1 = ((off_kv[seg_hi + 1] + TK - 1) // TK).astype(jnp.int32)

    grid_spec = pltpu.PrefetchScalarGridSpec(
        num_scalar_prefetch=3,
        grid=(nq,),
        in_specs=[
            pl.BlockSpec((TQ, d), lambda i: (i, 0)),
            pl.BlockSpec((n, d), lambda i: (0, 0)),
            pl.BlockSpec((1, n), lambda i: (0, 0)),
            pl.BlockSpec((1, n), lambda i: (0, 0)),
            pl.BlockSpec((d, d), lambda i: (0, 0)),
            pl.BlockSpec((1, d), lambda i: (0, 0)),
            pl.BlockSpec((d, d), lambda i: (0, 0)),
            pl.BlockSpec((1, d), lambda i: (0, 0)),
        ],
        out_specs=pl.BlockSpec((TQ, d), lambda i: (i, 0)),
    )
    return pl.pallas_call(
        _attn_mlp_kernel,
        grid_spec=grid_spec,
        out_shape=jax.ShapeDtypeStruct((n, d), jnp.float32),
        compiler_params=pltpu.CompilerParams(
            dimension_semantics=("arbitrary",),
        ),
        interpret=interpret,
    )(kv_t0, kv_t1, size.reshape(1), q, kv,
      qb.reshape(1, n), kvb.reshape(1, n), w1t, b1.reshape(1, d),
      w2t, b2.reshape(1, d))


def kernel(x_src, x_tar, W1, b1, W2, b2, batch_src, batch_tar,
           interpret=False):
    bs = batch_src.astype(jnp.int32)
    bt = batch_tar.astype(jnp.int32)
    size = jnp.where(bs[-1] == bt[-1], bs[-1] + 1,
                     jnp.minimum(bs[-1], bt[-1]) + 1).astype(jnp.int32)
    segs = jnp.arange(NUM_SEG + 1, dtype=jnp.int32)
    off_s = jnp.searchsorted(bs, segs).astype(jnp.int32)
    off_t = jnp.searchsorted(bt, segs).astype(jnp.int32)
    w1t = W1.T
    w2t = W2.T

    out_src = _cross_side(x_src, bs, x_tar, bt, off_t, size,
                          w1t, b1, w2t, b2, interpret=interpret)
    out_tar = _cross_side(x_tar, bt, x_src, bs, off_s, size,
                          w1t, b1, w2t, b2, interpret=interpret)
    return (out_tar, out_src)


# speedup vs baseline: 2.2875x; 2.2875x over previous
import functools

import jax
import jax.numpy as jnp
from jax.experimental import pallas as pl
from jax.experimental.pallas import tpu as pltpu

NUM_SEG = 8
TQ = 256
TK = 512
NEG = -1e30


def _attn_mlp_kernel(kv_t0_ref, kv_t1_ref, size_ref,
                     q_ref, kv_ref, qb_ref, kvb_ref,
                     w1t_ref, b1_ref, w2t_ref, b2_ref,
                     o_ref):
    i = pl.program_id(0)
    q = q_ref[...]
    qb = qb_ref[0, pl.ds(i * TQ, TQ)]
    qb_col = jnp.reshape(qb, (TQ, 1))

    t0 = kv_t0_ref[i]
    t1 = kv_t1_ref[i]

    m0 = jnp.full((TQ, 1), NEG, jnp.float32)
    l0 = jnp.zeros((TQ, 1), jnp.float32)
    acc0 = jnp.zeros_like(q)

    def body(t, carry):
        m, l, acc = carry
        kv = kv_ref[pl.ds(t * TK, TK), :]
        kvb = kvb_ref[0, pl.ds(t * TK, TK)]
        s = jax.lax.dot_general(q, kv, (((1,), (1,)), ((), ())),
                                preferred_element_type=jnp.float32)
        mask = qb_col == kvb[None, :]
        s = jnp.where(mask, s, NEG)
        m_new = jnp.maximum(m, jnp.max(s, axis=1, keepdims=True))
        p = jnp.where(mask, jnp.exp(s - m_new), 0.0)
        alpha = jnp.exp(m - m_new)
        l = l * alpha + jnp.sum(p, axis=1, keepdims=True)
        acc = acc * alpha + jax.lax.dot_general(
            p, kv, (((1,), (0,)), ((), ())),
            preferred_element_type=jnp.float32)
        return m_new, l, acc

    m, l, acc = jax.lax.fori_loop(t0, t1, body, (m0, l0, acc0))

    out = acc * jnp.where(l > 0.0, 1.0 / jnp.where(l > 0.0, l, 1.0), 0.0)
    res = out + q
    res = jnp.where(qb_col < size_ref[0], res, 0.0)

    h = jax.lax.dot_general(res, w1t_ref[...], (((1,), (0,)), ((), ())),
                            preferred_element_type=jnp.float32)
    h = jnp.maximum(h + b1_ref[...], 0.0)
    y = jax.lax.dot_general(h, w2t_ref[...], (((1,), (0,)), ((), ())),
                            preferred_element_type=jnp.float32)
    o_ref[...] = y + b2_ref[...] + res


@functools.partial(jax.jit, static_argnames=("interpret",))
def _cross_side(q, qb, kv, kvb, off_kv, size, w1t, b1, w2t, b2,
                interpret=False):
    n, d = q.shape
    nq = n // TQ
    qb2 = qb.reshape(nq, TQ)
    seg_lo = qb2[:, 0]
    seg_hi = qb2[:, -1]
    kv_t0 = (off_kv[seg_lo] // TK).astype(jnp.int32)
    kv_t1 = ((off_kv[seg_hi + 1] + TK - 1) // TK).astype(jnp.int32)

    grid_spec = pltpu.PrefetchScalarGridSpec(
        num_scalar_prefetch=3,
        grid=(nq,),
        in_specs=[
            pl.BlockSpec((TQ, d), lambda i, *_: (i, 0)),
            pl.BlockSpec((n, d), lambda i, *_: (0, 0)),
            pl.BlockSpec((1, n), lambda i, *_: (0, 0)),
            pl.BlockSpec((1, n), lambda i, *_: (0, 0)),
            pl.BlockSpec((d, d), lambda i, *_: (0, 0)),
            pl.BlockSpec((1, d), lambda i, *_: (0, 0)),
            pl.BlockSpec((d, d), lambda i, *_: (0, 0)),
            pl.BlockSpec((1, d), lambda i, *_: (0, 0)),
        ],
        out_specs=pl.BlockSpec((TQ, d), lambda i, *_: (i, 0)),
    )
    return pl.pallas_call(
        _attn_mlp_kernel,
        grid_spec=grid_spec,
        out_shape=jax.ShapeDtypeStruct((n, d), jnp.float32),
        compiler_params=pltpu.CompilerParams(
            dimension_semantics=("arbitrary",),
        ),
        interpret=interpret,
    )(kv_t0, kv_t1, size.reshape(1), q, kv,
      qb.reshape(1, n), kvb.reshape(1, n), w1t, b1.reshape(1, d),
      w2t, b2.reshape(1, d))


def kernel(x_src, x_tar, W1, b1, W2, b2, batch_src, batch_tar,
           interpret=False):
    bs = batch_src.astype(jnp.int32)
    bt = batch_tar.astype(jnp.int32)
    size = jnp.where(bs[-1] == bt[-1], bs[-1] + 1,
                     jnp.minimum(bs[-1], bt[-1]) + 1).astype(jnp.int32)
    segs = jnp.arange(NUM_SEG + 1, dtype=jnp.int32)
    off_s = jnp.searchsorted(bs, segs).astype(jnp.int32)
    off_t = jnp.searchsorted(bt, segs).astype(jnp.int32)
    w1t = W1.T
    w2t = W2.T

    out_src = _cross_side(x_src, bs, x_tar, bt, off_t, size,
                          w1t, b1, w2t, b2, interpret=interpret)
    out_tar = _cross_side(x_tar, bt, x_src, bs, off_s, size,
                          w1t, b1, w2t, b2, interpret=interpret)
    return (out_tar, out_src)
